# Initial kernel scaffold; baseline (speedup 1.0000x reference)
import functools

import jax
import jax.numpy as jnp
from jax import lax
from jax.experimental import pallas as pl
from jax.experimental.pallas import tpu as pltpu
from jax.experimental.pallas import tpu_sc as plsc

E = 160000
D = 256
L = 16
NW = 32
EPW = E // NW
C = 40
NCH = EPW // C


def _dot_body(h_hbm, src_hbm, dst_hbm, out_hbm, src_v, dst_v, u_v, v_v,
              out_v, sem):
    wid = lax.axis_index("s") * 2 + lax.axis_index("c")
    base = wid * EPW
    pltpu.sync_copy(src_hbm.at[wid], src_v)
    pltpu.sync_copy(dst_hbm.at[wid], dst_v)

    def chunk_body(j, _):
        cu = pltpu.async_copy(h_hbm.at[src_v.at[j]], u_v, sem)
        cv = pltpu.async_copy(h_hbm.at[dst_v.at[j]], v_v, sem)
        cu.wait()
        cv.wait()

        def edge_body(e, _):
            acc = u_v[e, pl.ds(0, L)] * v_v[e, pl.ds(0, L)]
            for k in range(1, D // L):
                acc = acc + u_v[e, pl.ds(k * L, L)] * v_v[e, pl.ds(k * L, L)]
            out_v[j * C + e] = jnp.sum(acc)
            return 0

        lax.fori_loop(0, C, edge_body, 0)
        return 0

    lax.fori_loop(0, NCH, chunk_body, 0)
    pltpu.sync_copy(out_v, out_hbm.at[pl.ds(base, EPW)])


_dot_kernel = functools.partial(
    pl.kernel,
    out_type=jax.ShapeDtypeStruct((E,), jnp.float32),
    mesh=plsc.VectorSubcoreMesh(core_axis_name="c", subcore_axis_name="s"),
    scratch_types=[
        pltpu.VMEM((NCH, C), jnp.int32),
        pltpu.VMEM((NCH, C), jnp.int32),
        pltpu.VMEM((C, D), jnp.float32),
        pltpu.VMEM((C, D), jnp.float32),
        pltpu.VMEM((EPW,), jnp.float32),
        pltpu.SemaphoreType.DMA,
    ],
)(_dot_body)


@jax.jit
def kernel(h, edge_index):
    src = edge_index[0].astype(jnp.int32).reshape(NW, NCH, C)
    dst = edge_index[1].astype(jnp.int32).reshape(NW, NCH, C)
    return _dot_kernel(h, src, dst)


# speedup vs baseline: 2.7849x; 2.7849x over previous
import functools

import jax
import jax.numpy as jnp
from jax import lax
from jax.experimental import pallas as pl
from jax.experimental.pallas import tpu as pltpu
from jax.experimental.pallas import tpu_sc as plsc

E = 160000
D = 256
L = 16
NW = 32
EPW = E // NW
C = 40
NCH = EPW // C


NG = (C + L - 1) // L


def _dot_body(h_hbm, src_hbm, dst_hbm, out_hbm, src_v, dst_v, u_v, v_v,
              out_v, sem):
    wid = lax.axis_index("s") * 2 + lax.axis_index("c")
    base = wid * EPW
    pltpu.sync_copy(src_hbm.at[wid], src_v)
    pltpu.sync_copy(dst_hbm.at[wid], dst_v)

    def chunk_body(j, _):
        cu = pltpu.async_copy(h_hbm.at[src_v.at[j]], u_v, sem)
        cv = pltpu.async_copy(h_hbm.at[dst_v.at[j]], v_v, sem)
        cu.wait()
        cv.wait()

        lane = lax.iota(jnp.int32, L)
        for g in range(NG):
            n = min(L, C - g * L)

            def edge_body(i, gvec, g=g):
                e = g * L + i
                p = [u_v[e, pl.ds(k * L, L)] * v_v[e, pl.ds(k * L, L)]
                     for k in range(D // L)]
                while len(p) > 1:
                    p = [p[a] + p[a + 1] for a in range(0, len(p), 2)]
                s = jnp.sum(p[0])
                return jnp.where(lane == i, s, gvec)

            gvec = lax.fori_loop(0, n, edge_body, jnp.zeros((L,), jnp.float32))
            out_v[pl.ds(j * C + g * L, L)] = gvec
        return 0

    lax.fori_loop(0, NCH, chunk_body, 0)
    pltpu.sync_copy(out_v.at[pl.ds(0, EPW)], out_hbm.at[pl.ds(base, EPW)])


_dot_kernel = functools.partial(
    pl.kernel,
    out_type=jax.ShapeDtypeStruct((E,), jnp.float32),
    mesh=plsc.VectorSubcoreMesh(core_axis_name="c", subcore_axis_name="s"),
    compiler_params=pltpu.CompilerParams(needs_layout_passes=False),
    scratch_types=[
        pltpu.VMEM((NCH, C), jnp.int32),
        pltpu.VMEM((NCH, C), jnp.int32),
        pltpu.VMEM((C, D), jnp.float32),
        pltpu.VMEM((C, D), jnp.float32),
        pltpu.VMEM((NCH * NG * L,), jnp.float32),
        pltpu.SemaphoreType.DMA,
    ],
)(_dot_body)


@jax.jit
def kernel(h, edge_index):
    src = edge_index[0].astype(jnp.int32).reshape(NW, NCH, C)
    dst = edge_index[1].astype(jnp.int32).reshape(NW, NCH, C)
    return _dot_kernel(h, src, dst)
